# baseline (speedup 1.0000x reference)
import functools

import jax
import jax.numpy as jnp
from jax import lax
from jax.experimental import pallas as pl
from jax.experimental.pallas import tpu as pltpu
from jax.experimental.pallas import tpu_sc as plsc

_RB = 128
_CB = 128
_K = 16


def _dot_t(a, b):
    return lax.dot_general(a, b, (((1,), (1,)), ((), ())),
                           preferred_element_type=jnp.float32)


def _pre_body(x_ref, w1_ref, b1_ref, wg_ref, h_ref, xw_ref):
    xb = x_ref[...]
    h = _dot_t(xb, w1_ref[...]) + b1_ref[...]
    h_ref[...] = h
    xw_ref[...] = _dot_t(h, wg_ref[...])


def _knn_body(b0, sinfo_ref, batch_r_ref, h_r_ref, h_ref, batch2d_ref,
              nbr_ref):
    b = pl.program_id(0) + b0
    cb0 = sinfo_ref[b, 0]
    ncb = sinfo_ref[b, 1]
    rows = b * _RB + lax.broadcasted_iota(jnp.int32, (_RB, 1), 0)
    batch_r = batch_r_ref[0]
    h_r = h_r_ref[...]
    sq_r = jnp.sum(h_r * h_r, axis=1, keepdims=True)

    def body(j, carry):
        best_d, best_i = carry
        hc = h_ref[pl.ds(j * _CB, _CB), :]
        sq_c = jnp.sum(hc * hc, axis=1)
        d = sq_r + sq_c - 2.0 * _dot_t(h_r, hc)
        batch_c = batch2d_ref[j]
        cols = j * _CB + lax.broadcasted_iota(jnp.int32, (1, _CB), 1)
        valid = (batch_r == batch_c[None, :]) & (rows != cols)
        d = jnp.where(valid, d, jnp.inf)
        colsf = (jnp.float32(j * _CB)
                 + lax.broadcasted_iota(jnp.int32, (1, _CB), 1).astype(jnp.float32))
        cand_d0 = jnp.concatenate([best_d, d], axis=1)
        cand_i0 = jnp.concatenate([best_i, jnp.broadcast_to(colsf, (_RB, _CB))],
                                  axis=1)
        nchains = 4
        rc = _RB // nchains
        outs = []
        for q in range(nchains):
            cand_d = cand_d0[q * rc:(q + 1) * rc]
            cand_i = cand_i0[q * rc:(q + 1) * rc]
            nd, ni = [], []
            for _ in range(_K):
                m = jnp.min(cand_d, axis=1, keepdims=True)
                onehot = cand_d == m
                sel = jnp.min(jnp.where(onehot, cand_i, jnp.float32(3e38)),
                              axis=1, keepdims=True)
                nd.append(m)
                ni.append(sel)
                cand_d = jnp.where(onehot, jnp.inf, cand_d)
            outs.append((jnp.concatenate(nd, axis=1),
                         jnp.concatenate(ni, axis=1)))
        return (jnp.concatenate([o[0] for o in outs], axis=0),
                jnp.concatenate([o[1] for o in outs], axis=0))

    init = (jnp.full((_RB, _K), jnp.inf, jnp.float32),
            jnp.zeros((_RB, _K), jnp.float32))
    _, best_i = lax.fori_loop(cb0, cb0 + ncb, body, init)
    npad = h_ref.shape[0]
    nbr_ref[...] = jnp.clip(best_i, 0, npad - 1).astype(jnp.int32)


def _gat_body(g3_ref, asrc_ref, adst_ref, bg_ref, w2_ref, b2_ref, w3_ref,
              b3_ref, w4_ref, b4_ref, out_ref):
    hid = w2_ref.shape[1]
    xw_self = g3_ref[:, _K, :]
    t_b = jnp.dot(xw_self, adst_ref[...], preferred_element_type=jnp.float32)
    s_nbr = jnp.concatenate(
        [jnp.dot(g3_ref[:, t, :], asrc_ref[...],
                 preferred_element_type=jnp.float32)
         for t in range(_K + 1)], axis=1)
    e = s_nbr + t_b
    e = jnp.where(e > 0, e, 0.2 * e)
    m = jnp.max(e, axis=1, keepdims=True)
    ee = jnp.exp(e - m)
    denom = jnp.sum(ee, axis=1, keepdims=True) + 1e-16
    alpha = ee / denom
    acc = jnp.zeros((_RB, hid), jnp.float32)
    for t in range(_K + 1):
        acc = acc + alpha[:, t:t + 1] * g3_ref[:, t, :]
    g = acc + bg_ref[...]
    h2 = jnp.maximum(_dot_t(g, w2_ref[...]) + b2_ref[...], 0.0)
    h3 = jnp.maximum(_dot_t(h2, w3_ref[...]) + b3_ref[...], 0.0)
    out_ref[...] = _dot_t(h3, w4_ref[...]) + b4_ref[...]


def _sc_gather(table, idx):
    nfo = plsc.get_sparse_core_info()
    nc, ns = nfo.num_cores, nfo.num_subcores
    nw = nc * ns
    m_total, d = idx.shape[0], table.shape[1]
    bpw = m_total // nw
    nchunk = bpw // 128
    mesh = plsc.VectorSubcoreMesh(core_axis_name="c", subcore_axis_name="s")

    nbuf = 4 if nchunk >= 4 else 2
    @functools.partial(
        pl.kernel, mesh=mesh,
        out_type=jax.ShapeDtypeStruct((m_total, d), jnp.float32),
        scratch_types=(
            [pltpu.VMEM((nchunk, 128), jnp.int32)]
            + [pltpu.VMEM((128, d), jnp.float32) for _ in range(nbuf)]
            + [pltpu.SemaphoreType.DMA for _ in range(2 * nbuf)]
        ),
    )
    def k(table_hbm, idx_hbm, out_hbm, idx_v, *rest):
        bufs = rest[:nbuf]
        gsems = rest[nbuf:2 * nbuf]
        ssems = rest[2 * nbuf:3 * nbuf]
        wid = lax.axis_index("s") * nc + lax.axis_index("c")
        pltpu.sync_copy(idx_hbm.at[wid], idx_v)
        gd = [None] * nbuf
        sd = [None] * nbuf
        for c in range(min(nbuf, nchunk)):
            gd[c] = pltpu.async_copy(table_hbm.at[idx_v.at[c]], bufs[c],
                                     gsems[c])
        for c in range(nchunk):
            cur = c % nbuf
            gd[cur].wait()
            sd[cur] = pltpu.async_copy(
                bufs[cur], out_hbm.at[pl.ds((wid * nchunk + c) * 128, 128)],
                ssems[cur])
            nx = c + nbuf
            if nx < nchunk:
                sd[cur].wait()
                gd[cur] = pltpu.async_copy(table_hbm.at[idx_v.at[nx]],
                                           bufs[cur], gsems[cur])
        for c in range(max(0, nchunk - nbuf), nchunk):
            sd[c % nbuf].wait()
    return k(table, idx.reshape(nw, nchunk, 128))


def kernel(x, batch, pairs_indices, pairs_labels, W1, b1, Wg, att_src, att_dst,
           bg, W2, b2, W3, b3, W4, b4):
    n, d_in = x.shape
    hid = W1.shape[0]
    nb = (n + _RB - 1) // _RB
    npad = nb * _RB

    xp = jnp.pad(x, ((0, npad - n), (0, 0)))
    batch_p = jnp.pad(batch.astype(jnp.int32), (0, npad - n),
                      constant_values=-1)

    idx_lo = jnp.minimum(jnp.arange(nb, dtype=jnp.int32) * _RB, n - 1)
    idx_hi = jnp.minimum(idx_lo + _RB - 1, n - 1)
    cs = jnp.searchsorted(batch, batch[idx_lo], side="left").astype(jnp.int32)
    ce = jnp.searchsorted(batch, batch[idx_hi], side="right").astype(jnp.int32)
    cb0 = cs // _CB
    ncb = (ce + _CB - 1) // _CB - cb0
    sinfo = jnp.stack([cb0, ncb], axis=1)

    f32 = jnp.float32
    grid = (nb,)
    row_spec = lambda lastdim: pl.BlockSpec((_RB, lastdim), lambda b_: (b_, 0))

    def whole(shape_arr):
        return pl.BlockSpec(shape_arr.shape, lambda b_: (0,) * shape_arr.ndim)

    h, xw = pl.pallas_call(
        _pre_body,
        grid=grid,
        in_specs=[row_spec(d_in), whole(W1), whole(b1.reshape(1, hid)),
                  whole(Wg)],
        out_specs=[row_spec(hid), row_spec(hid)],
        out_shape=[jax.ShapeDtypeStruct((npad, hid), f32),
                   jax.ShapeDtypeStruct((npad, hid), f32)],
    )(xp, W1, b1.reshape(1, hid), Wg)

    def half(b0, nblk):
        r0 = b0 * _RB
        nrows = nblk * _RB
        hgrid = (nblk,)
        hrow = lambda ld: pl.BlockSpec((_RB, ld), lambda b_: (b_ + b0, 0))
        nbr_h = pl.pallas_call(
            functools.partial(_knn_body, b0),
            grid=hgrid,
            in_specs=[pl.BlockSpec(memory_space=pltpu.SMEM),
                      pl.BlockSpec((1, _RB, 1), lambda b_: (b_ + b0, 0, 0)),
                      hrow(hid), whole(h),
                      whole(batch_p.reshape(nb, _RB))],
            out_specs=pl.BlockSpec((_RB, _K), lambda b_: (b_, 0)),
            out_shape=jax.ShapeDtypeStruct((nrows, _K), jnp.int32),
        )(sinfo, batch_p.reshape(nb, _RB, 1), h, h, batch_p.reshape(nb, _RB))

        idxg = jnp.concatenate(
            [nbr_h, (r0 + jnp.arange(nrows, dtype=jnp.int32))[:, None]],
            axis=1).reshape(-1)
        m_nodes = idxg.shape[0]
        m_pad = -m_nodes % (32 * 128)
        idxg = jnp.pad(idxg, (0, m_pad))
        gflat = _sc_gather(xw, idxg)
        g3 = gflat[:m_nodes].reshape(nrows, _K + 1, hid)

        return pl.pallas_call(
            _gat_body,
            grid=hgrid,
            in_specs=[pl.BlockSpec((_RB, _K + 1, hid), lambda b_: (b_, 0, 0)),
                      whole(att_src.reshape(hid, 1)),
                      whole(att_dst.reshape(hid, 1)),
                      whole(bg.reshape(1, hid)),
                      whole(W2), whole(b2.reshape(1, b2.shape[0])),
                      whole(W3), whole(b3.reshape(1, b3.shape[0])),
                      whole(W4), whole(b4.reshape(1, b4.shape[0]))],
            out_specs=pl.BlockSpec((_RB, W4.shape[0]), lambda b_: (b_, 0)),
            out_shape=jax.ShapeDtypeStruct((nrows, W4.shape[0]), f32),
        )(g3, att_src.reshape(hid, 1), att_dst.reshape(hid, 1),
          bg.reshape(1, hid), W2, b2.reshape(1, b2.shape[0]), W3,
          b3.reshape(1, b3.shape[0]), W4, b4.reshape(1, b4.shape[0]))

    bh = nb // 2
    hfin = jnp.concatenate([half(0, bh), half(bh, nb - bh)], axis=0)

    npairs = pairs_indices.shape[0]
    idx_flat = jnp.concatenate([pairs_indices[:, 0], pairs_indices[:, 1]]
                               ).astype(jnp.int32)
    pairs = _sc_gather(hfin, idx_flat)
    pair_embeddings = pairs.reshape(2, npairs, W4.shape[0])
    return pair_embeddings, pairs_labels


# speedup vs baseline: 1.3150x; 1.3150x over previous
import functools

import jax
import jax.numpy as jnp
from jax import lax
from jax.experimental import pallas as pl
from jax.experimental.pallas import tpu as pltpu
from jax.experimental.pallas import tpu_sc as plsc

_RB = 128
_CB = 256
_K = 16


def _dot_t(a, b):
    return lax.dot_general(a, b, (((1,), (1,)), ((), ())),
                           preferred_element_type=jnp.float32)


def _pre_body(x_ref, w1_ref, b1_ref, wg_ref, h_ref, xw_ref):
    xb = x_ref[...]
    h = _dot_t(xb, w1_ref[...]) + b1_ref[...]
    h_ref[...] = h
    xw_ref[...] = _dot_t(h, wg_ref[...])


def _knn_body(b0, sinfo_ref, batch_r_ref, h_r_ref, h_ref, batch2d_ref,
              nbr_ref):
    b = pl.program_id(0) + b0
    cb0 = sinfo_ref[b, 0]
    ncb = sinfo_ref[b, 1]
    rows = b * _RB + lax.broadcasted_iota(jnp.int32, (_RB, 1), 0)
    batch_r = batch_r_ref[0]
    h_r = h_r_ref[...]
    sq_r = jnp.sum(h_r * h_r, axis=1, keepdims=True)

    def body(j, carry):
        best_d, best_i = carry
        hc = h_ref[pl.ds(j * _CB, _CB), :]
        sq_c = jnp.sum(hc * hc, axis=1)
        d = sq_r + sq_c - 2.0 * _dot_t(h_r, hc)
        batch_c = batch2d_ref[j]
        cols = j * _CB + lax.broadcasted_iota(jnp.int32, (1, _CB), 1)
        valid = (batch_r == batch_c[None, :]) & (rows != cols)
        d = jnp.where(valid, d, jnp.inf)
        colsf = (jnp.float32(j * _CB)
                 + lax.broadcasted_iota(jnp.int32, (1, _CB), 1).astype(jnp.float32))
        cand_d0 = jnp.concatenate([best_d, d], axis=1)
        cand_i0 = jnp.concatenate([best_i, jnp.broadcast_to(colsf, (_RB, _CB))],
                                  axis=1)
        nchains = 4
        rc = _RB // nchains
        outs = []
        for q in range(nchains):
            cand_d = cand_d0[q * rc:(q + 1) * rc]
            cand_i = cand_i0[q * rc:(q + 1) * rc]
            nd, ni = [], []
            for _ in range(_K):
                m = jnp.min(cand_d, axis=1, keepdims=True)
                onehot = cand_d == m
                sel = jnp.min(jnp.where(onehot, cand_i, jnp.float32(3e38)),
                              axis=1, keepdims=True)
                nd.append(m)
                ni.append(sel)
                cand_d = jnp.where(onehot, jnp.inf, cand_d)
            outs.append((jnp.concatenate(nd, axis=1),
                         jnp.concatenate(ni, axis=1)))
        return (jnp.concatenate([o[0] for o in outs], axis=0),
                jnp.concatenate([o[1] for o in outs], axis=0))

    init = (jnp.full((_RB, _K), jnp.inf, jnp.float32),
            jnp.zeros((_RB, _K), jnp.float32))
    _, best_i = lax.fori_loop(cb0, cb0 + ncb, body, init)
    npad = h_ref.shape[0]
    nbr_ref[...] = jnp.clip(best_i, 0, npad - 1).astype(jnp.int32)


def _gat_body(g3_ref, asrc_ref, adst_ref, bg_ref, w2_ref, b2_ref, w3_ref,
              b3_ref, w4_ref, b4_ref, out_ref):
    hid = w2_ref.shape[1]
    xw_self = g3_ref[:, _K, :]
    t_b = jnp.dot(xw_self, adst_ref[...], preferred_element_type=jnp.float32)
    s_nbr = jnp.concatenate(
        [jnp.dot(g3_ref[:, t, :], asrc_ref[...],
                 preferred_element_type=jnp.float32)
         for t in range(_K + 1)], axis=1)
    e = s_nbr + t_b
    e = jnp.where(e > 0, e, 0.2 * e)
    m = jnp.max(e, axis=1, keepdims=True)
    ee = jnp.exp(e - m)
    denom = jnp.sum(ee, axis=1, keepdims=True) + 1e-16
    alpha = ee / denom
    acc = jnp.zeros((_RB, hid), jnp.float32)
    for t in range(_K + 1):
        acc = acc + alpha[:, t:t + 1] * g3_ref[:, t, :]
    g = acc + bg_ref[...]
    h2 = jnp.maximum(_dot_t(g, w2_ref[...]) + b2_ref[...], 0.0)
    h3 = jnp.maximum(_dot_t(h2, w3_ref[...]) + b3_ref[...], 0.0)
    out_ref[...] = _dot_t(h3, w4_ref[...]) + b4_ref[...]


def _sc_gather(table, idx):
    nfo = plsc.get_sparse_core_info()
    nc, ns = nfo.num_cores, nfo.num_subcores
    nw = nc * ns
    m_total, d = idx.shape[0], table.shape[1]
    bpw = m_total // nw
    nchunk = bpw // 128
    mesh = plsc.VectorSubcoreMesh(core_axis_name="c", subcore_axis_name="s")

    nbuf = 4 if nchunk >= 4 else 2
    @functools.partial(
        pl.kernel, mesh=mesh,
        out_type=jax.ShapeDtypeStruct((m_total, d), jnp.float32),
        scratch_types=(
            [pltpu.VMEM((nchunk, 128), jnp.int32)]
            + [pltpu.VMEM((128, d), jnp.float32) for _ in range(nbuf)]
            + [pltpu.SemaphoreType.DMA for _ in range(2 * nbuf)]
        ),
    )
    def k(table_hbm, idx_hbm, out_hbm, idx_v, *rest):
        bufs = rest[:nbuf]
        gsems = rest[nbuf:2 * nbuf]
        ssems = rest[2 * nbuf:3 * nbuf]
        wid = lax.axis_index("s") * nc + lax.axis_index("c")
        pltpu.sync_copy(idx_hbm.at[wid], idx_v)
        gd = [None] * nbuf
        sd = [None] * nbuf
        for c in range(min(nbuf, nchunk)):
            gd[c] = pltpu.async_copy(table_hbm.at[idx_v.at[c]], bufs[c],
                                     gsems[c])
        for c in range(nchunk):
            cur = c % nbuf
            gd[cur].wait()
            sd[cur] = pltpu.async_copy(
                bufs[cur], out_hbm.at[pl.ds((wid * nchunk + c) * 128, 128)],
                ssems[cur])
            nx = c + nbuf
            if nx < nchunk:
                sd[cur].wait()
                gd[cur] = pltpu.async_copy(table_hbm.at[idx_v.at[nx]],
                                           bufs[cur], gsems[cur])
        for c in range(max(0, nchunk - nbuf), nchunk):
            sd[c % nbuf].wait()
    return k(table, idx.reshape(nw, nchunk, 128))


def kernel(x, batch, pairs_indices, pairs_labels, W1, b1, Wg, att_src, att_dst,
           bg, W2, b2, W3, b3, W4, b4):
    n, d_in = x.shape
    hid = W1.shape[0]
    npad = n + (-n % _CB)
    nb = npad // _RB

    xp = jnp.pad(x, ((0, npad - n), (0, 0)))
    batch_p = jnp.pad(batch.astype(jnp.int32), (0, npad - n),
                      constant_values=-1)

    idx_lo = jnp.minimum(jnp.arange(nb, dtype=jnp.int32) * _RB, n - 1)
    idx_hi = jnp.minimum(idx_lo + _RB - 1, n - 1)
    cs = jnp.searchsorted(batch, batch[idx_lo], side="left").astype(jnp.int32)
    ce = jnp.searchsorted(batch, batch[idx_hi], side="right").astype(jnp.int32)
    cb0 = cs // _CB
    ncb = (ce + _CB - 1) // _CB - cb0
    sinfo = jnp.stack([cb0, ncb], axis=1)

    f32 = jnp.float32
    grid = (nb,)
    row_spec = lambda lastdim: pl.BlockSpec((_RB, lastdim), lambda b_: (b_, 0))

    def whole(shape_arr):
        return pl.BlockSpec(shape_arr.shape, lambda b_: (0,) * shape_arr.ndim)

    h, xw = pl.pallas_call(
        _pre_body,
        grid=grid,
        in_specs=[row_spec(d_in), whole(W1), whole(b1.reshape(1, hid)),
                  whole(Wg)],
        out_specs=[row_spec(hid), row_spec(hid)],
        out_shape=[jax.ShapeDtypeStruct((npad, hid), f32),
                   jax.ShapeDtypeStruct((npad, hid), f32)],
    )(xp, W1, b1.reshape(1, hid), Wg)

    nbr = pl.pallas_call(
        functools.partial(_knn_body, 0),
        grid=grid,
        in_specs=[pl.BlockSpec(memory_space=pltpu.SMEM),
                  pl.BlockSpec((1, _RB, 1), lambda b_: (b_, 0, 0)),
                  row_spec(hid), whole(h),
                  whole(batch_p.reshape(-1, _CB))],
        out_specs=pl.BlockSpec((_RB, _K), lambda b_: (b_, 0)),
        out_shape=jax.ShapeDtypeStruct((npad, _K), jnp.int32),
    )(sinfo, batch_p.reshape(nb, _RB, 1), h, h, batch_p.reshape(-1, _CB))

    idxg = jnp.concatenate(
        [nbr, jnp.arange(npad, dtype=jnp.int32)[:, None]], axis=1).reshape(-1)
    m_nodes = idxg.shape[0]
    m_pad = -m_nodes % (32 * 128)
    idxg = jnp.pad(idxg, (0, m_pad))
    gflat = _sc_gather(xw, idxg)
    g3 = gflat[:m_nodes].reshape(npad, _K + 1, hid)

    hfin = pl.pallas_call(
        _gat_body,
        grid=grid,
        in_specs=[pl.BlockSpec((_RB, _K + 1, hid), lambda b_: (b_, 0, 0)),
                  whole(att_src.reshape(hid, 1)), whole(att_dst.reshape(hid, 1)),
                  whole(bg.reshape(1, hid)),
                  whole(W2), whole(b2.reshape(1, b2.shape[0])),
                  whole(W3), whole(b3.reshape(1, b3.shape[0])),
                  whole(W4), whole(b4.reshape(1, b4.shape[0]))],
        out_specs=row_spec(W4.shape[0]),
        out_shape=jax.ShapeDtypeStruct((npad, W4.shape[0]), f32),
    )(g3, att_src.reshape(hid, 1), att_dst.reshape(hid, 1), bg.reshape(1, hid),
      W2, b2.reshape(1, b2.shape[0]), W3, b3.reshape(1, b3.shape[0]), W4,
      b4.reshape(1, b4.shape[0]))

    npairs = pairs_indices.shape[0]
    idx_flat = jnp.concatenate([pairs_indices[:, 0], pairs_indices[:, 1]]
                               ).astype(jnp.int32)
    pairs = _sc_gather(hfin, idx_flat)
    pair_embeddings = pairs.reshape(2, npairs, W4.shape[0])
    return pair_embeddings, pairs_labels


# speedup vs baseline: 1.6271x; 1.2374x over previous
import functools

import jax
import jax.numpy as jnp
from jax import lax
from jax.experimental import pallas as pl
from jax.experimental.pallas import tpu as pltpu
from jax.experimental.pallas import tpu_sc as plsc

_RB = 128
_CB = 512
_K = 16


def _dot_t(a, b):
    return lax.dot_general(a, b, (((1,), (1,)), ((), ())),
                           preferred_element_type=jnp.float32)


def _pre_body(x_ref, w1_ref, b1_ref, wg_ref, h_ref, xw_ref):
    xb = x_ref[...]
    h = _dot_t(xb, w1_ref[...]) + b1_ref[...]
    h_ref[...] = h
    xw_ref[...] = _dot_t(h, wg_ref[...])


def _knn_body(b0, sinfo_ref, batch_r_ref, h_r_ref, h_ref, batch2d_ref,
              nbr_ref):
    b = pl.program_id(0) + b0
    cb0 = sinfo_ref[b, 0]
    ncb = sinfo_ref[b, 1]
    rows = b * _RB + lax.broadcasted_iota(jnp.int32, (_RB, 1), 0)
    batch_r = batch_r_ref[0]
    h_r = h_r_ref[...]
    sq_r = jnp.sum(h_r * h_r, axis=1, keepdims=True)

    def body(j, carry):
        best_d, best_i = carry
        hc = h_ref[pl.ds(j * _CB, _CB), :]
        sq_c = jnp.sum(hc * hc, axis=1)
        d = sq_r + sq_c - 2.0 * _dot_t(h_r, hc)
        batch_c = batch2d_ref[j]
        cols = j * _CB + lax.broadcasted_iota(jnp.int32, (1, _CB), 1)
        valid = (batch_r == batch_c[None, :]) & (rows != cols)
        d = jnp.where(valid, d, jnp.inf)
        colsf = (jnp.float32(j * _CB)
                 + lax.broadcasted_iota(jnp.int32, (1, _CB), 1).astype(jnp.float32))
        cand_d0 = jnp.concatenate([best_d, d], axis=1)
        cand_i0 = jnp.concatenate([best_i, jnp.broadcast_to(colsf, (_RB, _CB))],
                                  axis=1)
        nchains = 4
        rc = _RB // nchains
        outs = []
        for q in range(nchains):
            cand_d = cand_d0[q * rc:(q + 1) * rc]
            cand_i = cand_i0[q * rc:(q + 1) * rc]
            nd, ni = [], []
            for _ in range(_K):
                m = jnp.min(cand_d, axis=1, keepdims=True)
                onehot = cand_d == m
                sel = jnp.min(jnp.where(onehot, cand_i, jnp.float32(3e38)),
                              axis=1, keepdims=True)
                nd.append(m)
                ni.append(sel)
                cand_d = jnp.where(onehot, jnp.inf, cand_d)
            outs.append((jnp.concatenate(nd, axis=1),
                         jnp.concatenate(ni, axis=1)))
        return (jnp.concatenate([o[0] for o in outs], axis=0),
                jnp.concatenate([o[1] for o in outs], axis=0))

    init = (jnp.full((_RB, _K), jnp.inf, jnp.float32),
            jnp.zeros((_RB, _K), jnp.float32))
    _, best_i = lax.fori_loop(cb0, cb0 + ncb, body, init)
    npad = h_ref.shape[0]
    nbr_ref[...] = jnp.clip(best_i, 0, npad - 1).astype(jnp.int32)


def _gat_body(g3_ref, asrc_ref, adst_ref, bg_ref, w2_ref, b2_ref, w3_ref,
              b3_ref, w4_ref, b4_ref, out_ref):
    hid = w2_ref.shape[1]
    xw_self = g3_ref[:, _K, :]
    t_b = jnp.dot(xw_self, adst_ref[...], preferred_element_type=jnp.float32)
    s_nbr = jnp.concatenate(
        [jnp.dot(g3_ref[:, t, :], asrc_ref[...],
                 preferred_element_type=jnp.float32)
         for t in range(_K + 1)], axis=1)
    e = s_nbr + t_b
    e = jnp.where(e > 0, e, 0.2 * e)
    m = jnp.max(e, axis=1, keepdims=True)
    ee = jnp.exp(e - m)
    denom = jnp.sum(ee, axis=1, keepdims=True) + 1e-16
    alpha = ee / denom
    acc = jnp.zeros((_RB, hid), jnp.float32)
    for t in range(_K + 1):
        acc = acc + alpha[:, t:t + 1] * g3_ref[:, t, :]
    g = acc + bg_ref[...]
    h2 = jnp.maximum(_dot_t(g, w2_ref[...]) + b2_ref[...], 0.0)
    h3 = jnp.maximum(_dot_t(h2, w3_ref[...]) + b3_ref[...], 0.0)
    out_ref[...] = _dot_t(h3, w4_ref[...]) + b4_ref[...]


def _sc_gather(table, idx):
    nfo = plsc.get_sparse_core_info()
    nc, ns = nfo.num_cores, nfo.num_subcores
    nw = nc * ns
    m_total, d = idx.shape[0], table.shape[1]
    bpw = m_total // nw
    nchunk = bpw // 128
    mesh = plsc.VectorSubcoreMesh(core_axis_name="c", subcore_axis_name="s")

    nbuf = 4 if nchunk >= 4 else 2
    @functools.partial(
        pl.kernel, mesh=mesh,
        out_type=jax.ShapeDtypeStruct((m_total, d), jnp.float32),
        scratch_types=(
            [pltpu.VMEM((nchunk, 128), jnp.int32)]
            + [pltpu.VMEM((128, d), jnp.float32) for _ in range(nbuf)]
            + [pltpu.SemaphoreType.DMA for _ in range(2 * nbuf)]
        ),
    )
    def k(table_hbm, idx_hbm, out_hbm, idx_v, *rest):
        bufs = rest[:nbuf]
        gsems = rest[nbuf:2 * nbuf]
        ssems = rest[2 * nbuf:3 * nbuf]
        wid = lax.axis_index("s") * nc + lax.axis_index("c")
        pltpu.sync_copy(idx_hbm.at[wid], idx_v)
        gd = [None] * nbuf
        sd = [None] * nbuf
        for c in range(min(nbuf, nchunk)):
            gd[c] = pltpu.async_copy(table_hbm.at[idx_v.at[c]], bufs[c],
                                     gsems[c])
        for c in range(nchunk):
            cur = c % nbuf
            gd[cur].wait()
            sd[cur] = pltpu.async_copy(
                bufs[cur], out_hbm.at[pl.ds((wid * nchunk + c) * 128, 128)],
                ssems[cur])
            nx = c + nbuf
            if nx < nchunk:
                sd[cur].wait()
                gd[cur] = pltpu.async_copy(table_hbm.at[idx_v.at[nx]],
                                           bufs[cur], gsems[cur])
        for c in range(max(0, nchunk - nbuf), nchunk):
            sd[c % nbuf].wait()
    return k(table, idx.reshape(nw, nchunk, 128))


def kernel(x, batch, pairs_indices, pairs_labels, W1, b1, Wg, att_src, att_dst,
           bg, W2, b2, W3, b3, W4, b4):
    n, d_in = x.shape
    hid = W1.shape[0]
    npad = n + (-n % _CB)
    nb = npad // _RB

    xp = jnp.pad(x, ((0, npad - n), (0, 0)))
    batch_p = jnp.pad(batch.astype(jnp.int32), (0, npad - n),
                      constant_values=-1)

    idx_lo = jnp.minimum(jnp.arange(nb, dtype=jnp.int32) * _RB, n - 1)
    idx_hi = jnp.minimum(idx_lo + _RB - 1, n - 1)
    cs = jnp.searchsorted(batch, batch[idx_lo], side="left").astype(jnp.int32)
    ce = jnp.searchsorted(batch, batch[idx_hi], side="right").astype(jnp.int32)
    cb0 = cs // _CB
    ncb = (ce + _CB - 1) // _CB - cb0
    sinfo = jnp.stack([cb0, ncb], axis=1)

    f32 = jnp.float32
    grid = (nb,)
    row_spec = lambda lastdim: pl.BlockSpec((_RB, lastdim), lambda b_: (b_, 0))

    def whole(shape_arr):
        return pl.BlockSpec(shape_arr.shape, lambda b_: (0,) * shape_arr.ndim)

    h, xw = pl.pallas_call(
        _pre_body,
        grid=grid,
        in_specs=[row_spec(d_in), whole(W1), whole(b1.reshape(1, hid)),
                  whole(Wg)],
        out_specs=[row_spec(hid), row_spec(hid)],
        out_shape=[jax.ShapeDtypeStruct((npad, hid), f32),
                   jax.ShapeDtypeStruct((npad, hid), f32)],
    )(xp, W1, b1.reshape(1, hid), Wg)

    nbr = pl.pallas_call(
        functools.partial(_knn_body, 0),
        grid=grid,
        in_specs=[pl.BlockSpec(memory_space=pltpu.SMEM),
                  pl.BlockSpec((1, _RB, 1), lambda b_: (b_, 0, 0)),
                  row_spec(hid), whole(h),
                  whole(batch_p.reshape(-1, _CB))],
        out_specs=pl.BlockSpec((_RB, _K), lambda b_: (b_, 0)),
        out_shape=jax.ShapeDtypeStruct((npad, _K), jnp.int32),
    )(sinfo, batch_p.reshape(nb, _RB, 1), h, h, batch_p.reshape(-1, _CB))

    idxg = jnp.concatenate(
        [nbr, jnp.arange(npad, dtype=jnp.int32)[:, None]], axis=1).reshape(-1)
    m_nodes = idxg.shape[0]
    m_pad = -m_nodes % (32 * 128)
    idxg = jnp.pad(idxg, (0, m_pad))
    gflat = _sc_gather(xw, idxg)
    g3 = gflat[:m_nodes].reshape(npad, _K + 1, hid)

    hfin = pl.pallas_call(
        _gat_body,
        grid=grid,
        in_specs=[pl.BlockSpec((_RB, _K + 1, hid), lambda b_: (b_, 0, 0)),
                  whole(att_src.reshape(hid, 1)), whole(att_dst.reshape(hid, 1)),
                  whole(bg.reshape(1, hid)),
                  whole(W2), whole(b2.reshape(1, b2.shape[0])),
                  whole(W3), whole(b3.reshape(1, b3.shape[0])),
                  whole(W4), whole(b4.reshape(1, b4.shape[0]))],
        out_specs=row_spec(W4.shape[0]),
        out_shape=jax.ShapeDtypeStruct((npad, W4.shape[0]), f32),
    )(g3, att_src.reshape(hid, 1), att_dst.reshape(hid, 1), bg.reshape(1, hid),
      W2, b2.reshape(1, b2.shape[0]), W3, b3.reshape(1, b3.shape[0]), W4,
      b4.reshape(1, b4.shape[0]))

    npairs = pairs_indices.shape[0]
    idx_flat = jnp.concatenate([pairs_indices[:, 0], pairs_indices[:, 1]]
                               ).astype(jnp.int32)
    pairs = _sc_gather(hfin, idx_flat)
    pair_embeddings = pairs.reshape(2, npairs, W4.shape[0])
    return pair_embeddings, pairs_labels


# speedup vs baseline: 1.7280x; 1.0620x over previous
import functools

import jax
import jax.numpy as jnp
from jax import lax
from jax.experimental import pallas as pl
from jax.experimental.pallas import tpu as pltpu
from jax.experimental.pallas import tpu_sc as plsc

_RB = 128
_CB = 1024
_K = 16


def _dot_t(a, b):
    return lax.dot_general(a, b, (((1,), (1,)), ((), ())),
                           preferred_element_type=jnp.float32)


def _pre_body(x_ref, w1_ref, b1_ref, wg_ref, h_ref, xw_ref):
    xb = x_ref[...]
    h = _dot_t(xb, w1_ref[...]) + b1_ref[...]
    h_ref[...] = h
    xw_ref[...] = _dot_t(h, wg_ref[...])


def _knn_body(b0, sinfo_ref, batch_r_ref, h_r_ref, h_ref, batch2d_ref,
              nbr_ref):
    b = pl.program_id(0) + b0
    cb0 = sinfo_ref[b, 0]
    ncb = sinfo_ref[b, 1]
    rows = b * _RB + lax.broadcasted_iota(jnp.int32, (_RB, 1), 0)
    batch_r = batch_r_ref[0]
    h_r = h_r_ref[...]
    sq_r = jnp.sum(h_r * h_r, axis=1, keepdims=True)

    def body(j, carry):
        best_d, best_i = carry
        hc = h_ref[pl.ds(j * _CB, _CB), :]
        sq_c = jnp.sum(hc * hc, axis=1)
        d = sq_r + sq_c - 2.0 * _dot_t(h_r, hc)
        batch_c = batch2d_ref[j]
        cols = j * _CB + lax.broadcasted_iota(jnp.int32, (1, _CB), 1)
        valid = (batch_r == batch_c[None, :]) & (rows != cols)
        d = jnp.where(valid, d, jnp.inf)
        colsf = (jnp.float32(j * _CB)
                 + lax.broadcasted_iota(jnp.int32, (1, _CB), 1).astype(jnp.float32))
        cand_d0 = jnp.concatenate([best_d, d], axis=1)
        cand_i0 = jnp.concatenate([best_i, jnp.broadcast_to(colsf, (_RB, _CB))],
                                  axis=1)
        nchains = 4
        rc = _RB // nchains
        outs = []
        for q in range(nchains):
            cand_d = cand_d0[q * rc:(q + 1) * rc]
            cand_i = cand_i0[q * rc:(q + 1) * rc]
            nd, ni = [], []
            for _ in range(_K):
                m = jnp.min(cand_d, axis=1, keepdims=True)
                onehot = cand_d == m
                sel = jnp.min(jnp.where(onehot, cand_i, jnp.float32(3e38)),
                              axis=1, keepdims=True)
                nd.append(m)
                ni.append(sel)
                cand_d = jnp.where(onehot, jnp.inf, cand_d)
            outs.append((jnp.concatenate(nd, axis=1),
                         jnp.concatenate(ni, axis=1)))
        return (jnp.concatenate([o[0] for o in outs], axis=0),
                jnp.concatenate([o[1] for o in outs], axis=0))

    init = (jnp.full((_RB, _K), jnp.inf, jnp.float32),
            jnp.zeros((_RB, _K), jnp.float32))
    _, best_i = lax.fori_loop(cb0, cb0 + ncb, body, init)
    npad = h_ref.shape[0]
    nbr_ref[...] = jnp.clip(best_i, 0, npad - 1).astype(jnp.int32)


def _gat_body(g3_ref, asrc_ref, adst_ref, bg_ref, w2_ref, b2_ref, w3_ref,
              b3_ref, w4_ref, b4_ref, out_ref):
    hid = w2_ref.shape[1]
    xw_self = g3_ref[:, _K, :]
    t_b = jnp.dot(xw_self, adst_ref[...], preferred_element_type=jnp.float32)
    s_nbr = jnp.concatenate(
        [jnp.dot(g3_ref[:, t, :], asrc_ref[...],
                 preferred_element_type=jnp.float32)
         for t in range(_K + 1)], axis=1)
    e = s_nbr + t_b
    e = jnp.where(e > 0, e, 0.2 * e)
    m = jnp.max(e, axis=1, keepdims=True)
    ee = jnp.exp(e - m)
    denom = jnp.sum(ee, axis=1, keepdims=True) + 1e-16
    alpha = ee / denom
    acc = jnp.zeros((_RB, hid), jnp.float32)
    for t in range(_K + 1):
        acc = acc + alpha[:, t:t + 1] * g3_ref[:, t, :]
    g = acc + bg_ref[...]
    h2 = jnp.maximum(_dot_t(g, w2_ref[...]) + b2_ref[...], 0.0)
    h3 = jnp.maximum(_dot_t(h2, w3_ref[...]) + b3_ref[...], 0.0)
    out_ref[...] = _dot_t(h3, w4_ref[...]) + b4_ref[...]


def _sc_gather(table, idx):
    nfo = plsc.get_sparse_core_info()
    nc, ns = nfo.num_cores, nfo.num_subcores
    nw = nc * ns
    m_total, d = idx.shape[0], table.shape[1]
    bpw = m_total // nw
    nchunk = bpw // 128
    mesh = plsc.VectorSubcoreMesh(core_axis_name="c", subcore_axis_name="s")

    nbuf = 4 if nchunk >= 4 else 2
    @functools.partial(
        pl.kernel, mesh=mesh,
        out_type=jax.ShapeDtypeStruct((m_total, d), jnp.float32),
        scratch_types=(
            [pltpu.VMEM((nchunk, 128), jnp.int32)]
            + [pltpu.VMEM((128, d), jnp.float32) for _ in range(nbuf)]
            + [pltpu.SemaphoreType.DMA for _ in range(2 * nbuf)]
        ),
    )
    def k(table_hbm, idx_hbm, out_hbm, idx_v, *rest):
        bufs = rest[:nbuf]
        gsems = rest[nbuf:2 * nbuf]
        ssems = rest[2 * nbuf:3 * nbuf]
        wid = lax.axis_index("s") * nc + lax.axis_index("c")
        pltpu.sync_copy(idx_hbm.at[wid], idx_v)
        gd = [None] * nbuf
        sd = [None] * nbuf
        for c in range(min(nbuf, nchunk)):
            gd[c] = pltpu.async_copy(table_hbm.at[idx_v.at[c]], bufs[c],
                                     gsems[c])
        for c in range(nchunk):
            cur = c % nbuf
            gd[cur].wait()
            sd[cur] = pltpu.async_copy(
                bufs[cur], out_hbm.at[pl.ds((wid * nchunk + c) * 128, 128)],
                ssems[cur])
            nx = c + nbuf
            if nx < nchunk:
                sd[cur].wait()
                gd[cur] = pltpu.async_copy(table_hbm.at[idx_v.at[nx]],
                                           bufs[cur], gsems[cur])
        for c in range(max(0, nchunk - nbuf), nchunk):
            sd[c % nbuf].wait()
    return k(table, idx.reshape(nw, nchunk, 128))


def kernel(x, batch, pairs_indices, pairs_labels, W1, b1, Wg, att_src, att_dst,
           bg, W2, b2, W3, b3, W4, b4):
    n, d_in = x.shape
    hid = W1.shape[0]
    npad = n + (-n % _CB)
    nb = npad // _RB

    xp = jnp.pad(x, ((0, npad - n), (0, 0)))
    batch_p = jnp.pad(batch.astype(jnp.int32), (0, npad - n),
                      constant_values=-1)

    idx_lo = jnp.minimum(jnp.arange(nb, dtype=jnp.int32) * _RB, n - 1)
    idx_hi = jnp.minimum(idx_lo + _RB - 1, n - 1)
    cs = jnp.searchsorted(batch, batch[idx_lo], side="left").astype(jnp.int32)
    ce = jnp.searchsorted(batch, batch[idx_hi], side="right").astype(jnp.int32)
    cb0 = cs // _CB
    ncb = (ce + _CB - 1) // _CB - cb0
    sinfo = jnp.stack([cb0, ncb], axis=1)

    f32 = jnp.float32
    grid = (nb,)
    row_spec = lambda lastdim: pl.BlockSpec((_RB, lastdim), lambda b_: (b_, 0))

    def whole(shape_arr):
        return pl.BlockSpec(shape_arr.shape, lambda b_: (0,) * shape_arr.ndim)

    h, xw = pl.pallas_call(
        _pre_body,
        grid=grid,
        in_specs=[row_spec(d_in), whole(W1), whole(b1.reshape(1, hid)),
                  whole(Wg)],
        out_specs=[row_spec(hid), row_spec(hid)],
        out_shape=[jax.ShapeDtypeStruct((npad, hid), f32),
                   jax.ShapeDtypeStruct((npad, hid), f32)],
    )(xp, W1, b1.reshape(1, hid), Wg)

    nbr = pl.pallas_call(
        functools.partial(_knn_body, 0),
        grid=grid,
        in_specs=[pl.BlockSpec(memory_space=pltpu.SMEM),
                  pl.BlockSpec((1, _RB, 1), lambda b_: (b_, 0, 0)),
                  row_spec(hid), whole(h),
                  whole(batch_p.reshape(-1, _CB))],
        out_specs=pl.BlockSpec((_RB, _K), lambda b_: (b_, 0)),
        out_shape=jax.ShapeDtypeStruct((npad, _K), jnp.int32),
    )(sinfo, batch_p.reshape(nb, _RB, 1), h, h, batch_p.reshape(-1, _CB))

    idxg = jnp.concatenate(
        [nbr, jnp.arange(npad, dtype=jnp.int32)[:, None]], axis=1).reshape(-1)
    m_nodes = idxg.shape[0]
    m_pad = -m_nodes % (32 * 128)
    idxg = jnp.pad(idxg, (0, m_pad))
    gflat = _sc_gather(xw, idxg)
    g3 = gflat[:m_nodes].reshape(npad, _K + 1, hid)

    hfin = pl.pallas_call(
        _gat_body,
        grid=grid,
        in_specs=[pl.BlockSpec((_RB, _K + 1, hid), lambda b_: (b_, 0, 0)),
                  whole(att_src.reshape(hid, 1)), whole(att_dst.reshape(hid, 1)),
                  whole(bg.reshape(1, hid)),
                  whole(W2), whole(b2.reshape(1, b2.shape[0])),
                  whole(W3), whole(b3.reshape(1, b3.shape[0])),
                  whole(W4), whole(b4.reshape(1, b4.shape[0]))],
        out_specs=row_spec(W4.shape[0]),
        out_shape=jax.ShapeDtypeStruct((npad, W4.shape[0]), f32),
    )(g3, att_src.reshape(hid, 1), att_dst.reshape(hid, 1), bg.reshape(1, hid),
      W2, b2.reshape(1, b2.shape[0]), W3, b3.reshape(1, b3.shape[0]), W4,
      b4.reshape(1, b4.shape[0]))

    npairs = pairs_indices.shape[0]
    idx_flat = jnp.concatenate([pairs_indices[:, 0], pairs_indices[:, 1]]
                               ).astype(jnp.int32)
    pairs = _sc_gather(hfin, idx_flat)
    pair_embeddings = pairs.reshape(2, npairs, W4.shape[0])
    return pair_embeddings, pairs_labels
